# Initial kernel scaffold; baseline (speedup 1.0000x reference)
import jax
import jax.numpy as jnp
from jax.experimental import pallas as pl


def kernel(x, edge_index, W_self1, W_neigh1, b1, W_self2, W_neigh2, b2):
    raise NotImplementedError("write your pallas kernel here")


# speedup vs baseline: 5.1163x; 5.1163x over previous
import functools

import jax
import jax.numpy as jnp
from jax import lax
from jax.experimental import pallas as pl
from jax.experimental.pallas import tpu as pltpu
from jax.experimental.pallas import tpu_sc as plsc

N_NODES = 10000
D = 128

NC = 2
NS = 16
NW = NC * NS
K = 128

AGG_ROWS = 10112
DEG_LEN = 10240


def _sc_segment_sum(feat, src_lin, dst_chk, zeros2d, zeros1d, ones_row,
                    n_chunks, with_deg):
  mesh = plsc.VectorSubcoreMesh(core_axis_name="c", subcore_axis_name="s")

  out_type = [jax.ShapeDtypeStruct((NC, AGG_ROWS, D), jnp.float32)]
  if with_deg:
    out_type.append(jax.ShapeDtypeStruct((NC, DEG_LEN), jnp.float32))

  scratch = dict(
      src_v=pltpu.VMEM((n_chunks * K,), jnp.int32),
      dst_v=pltpu.VMEM((n_chunks, K), jnp.int32),
      rows_v=pltpu.VMEM((K, D), jnp.float32),
      ones_v=pltpu.VMEM((K,), jnp.float32),
      agg_sh=pltpu.VMEM_SHARED((AGG_ROWS, D), jnp.float32),
      deg_sh=pltpu.VMEM_SHARED((DEG_LEN,), jnp.float32),
      gsem=pltpu.SemaphoreType.DMA,
  )

  def body(feat_hbm, src_hbm, dst_hbm, z2_hbm, z1_hbm, ones_hbm,
           *outs, src_v, dst_v, rows_v, ones_v, agg_sh, deg_sh, gsem):
    if with_deg:
      agg_out, deg_out = outs
    else:
      (agg_out,) = outs
    c = lax.axis_index("c")
    s = lax.axis_index("s")
    wid = c * NS + s

    pltpu.sync_copy(src_hbm.at[wid], src_v)
    pltpu.sync_copy(dst_hbm.at[wid], dst_v)
    if with_deg:
      pltpu.sync_copy(ones_hbm, ones_v)

    pltpu.sync_copy(z2_hbm, agg_sh.at[pl.ds(s * (AGG_ROWS // NS), AGG_ROWS // NS)])
    if with_deg:
      pltpu.sync_copy(z1_hbm, deg_sh.at[pl.ds(s * (DEG_LEN // NS), DEG_LEN // NS)])
    plsc.subcore_barrier()

    def chunk(j, carry):
      idx = src_v.at[pl.ds(j * K, K)]
      pltpu.async_copy(feat_hbm.at[idx], rows_v, gsem).wait()
      pltpu.sync_copy(rows_v, agg_sh.at[dst_v.at[j]], add=True)
      if with_deg:
        pltpu.sync_copy(ones_v, deg_sh.at[dst_v.at[j]], add=True)
      return carry

    lax.fori_loop(0, n_chunks, chunk, 0)
    plsc.subcore_barrier()

    ar = AGG_ROWS // NS
    pltpu.sync_copy(agg_sh.at[pl.ds(s * ar, ar)],
                    agg_out.at[c, pl.ds(s * ar, ar)])
    if with_deg:
      dr = DEG_LEN // NS
      pltpu.sync_copy(deg_sh.at[pl.ds(s * dr, dr)],
                      deg_out.at[c, pl.ds(s * dr, dr)])

  run = pl.kernel(body, out_type=tuple(out_type), mesh=mesh,
                  scratch_types=scratch)
  return run(feat, src_lin, dst_chk, zeros2d, zeros1d, ones_row)


def _tc_layer(x, a0, a1, degT, w_self, w_neigh, b2d, apply_relu):
  n = x.shape[0]
  br = 2000

  def body(x_r, a0_r, a1_r, dg_r, ws_r, wn_r, b_r, o_r):
    agg = a0_r[...] + a1_r[...]
    deg = dg_r[..., 0] + dg_r[..., 1]
    invd = 1.0 / jnp.maximum(deg, 1.0)
    mean = agg * invd[:, None]
    h = (jnp.dot(x_r[...], ws_r[...], preferred_element_type=jnp.float32)
         + jnp.dot(mean, wn_r[...], preferred_element_type=jnp.float32)
         + b_r[...])
    o_r[...] = jnp.maximum(h, 0.0) if apply_relu else h

  grid = (n // br,)
  row_spec = pl.BlockSpec((br, D), lambda i: (i, 0))
  full = lambda shape: pl.BlockSpec(shape, lambda i: (0,) * len(shape))
  return pl.pallas_call(
      body,
      grid=grid,
      in_specs=[row_spec, row_spec, row_spec,
                pl.BlockSpec((br, 2), lambda i: (i, 0)),
                full((D, D)), full((D, D)), full((1, D))],
      out_specs=row_spec,
      out_shape=jax.ShapeDtypeStruct((n, D), jnp.float32),
  )(x, a0, a1, degT, w_self, w_neigh, b2d)


def kernel(x, edge_index, W_self1, W_neigh1, b1, W_self2, W_neigh2, b2):
  n, d = x.shape
  e = edge_index.shape[1]
  src = edge_index[0].astype(jnp.int32)
  dst = edge_index[1].astype(jnp.int32)

  n_chunks = -(-e // (NW * K))
  e_pad = NW * n_chunks * K
  src = jnp.pad(src, (0, e_pad - e))
  dst = jnp.pad(dst, (0, e_pad - e), constant_values=n)
  src_lin = src.reshape(NW, n_chunks * K)
  dst_chk = dst.reshape(NW, n_chunks, K)

  zeros2d = jnp.zeros((AGG_ROWS // NS, D), jnp.float32)
  zeros1d = jnp.zeros((DEG_LEN // NS,), jnp.float32)
  ones_row = jnp.ones((K,), jnp.float32)

  agg1, deg = _sc_segment_sum(x, src_lin, dst_chk, zeros2d, zeros1d,
                              ones_row, n_chunks, with_deg=True)
  degT = jnp.stack([deg[0, :n], deg[1, :n]], axis=1)
  b1_2d = b1.reshape(1, d)
  h = _tc_layer(x, agg1[0, :n], agg1[1, :n], degT, W_self1, W_neigh1,
                b1_2d, apply_relu=True)

  (agg2,) = _sc_segment_sum(h, src_lin, dst_chk, zeros2d, zeros1d,
                            ones_row, n_chunks, with_deg=False)
  out = _tc_layer(h, agg2[0, :n], agg2[1, :n], degT, W_self2, W_neigh2,
                  b2.reshape(1, d), apply_relu=False)
  return out
